# baseline (speedup 1.0000x reference)
import functools

import jax
import jax.numpy as jnp
from jax import lax
from jax.experimental import pallas as pl
from jax.experimental.pallas import tpu as pltpu
from jax.experimental.pallas import tpu_sc as plsc

N = 10000
E = 320000
D = 128

NP = 10240
TRASH = 10000
K = 128
TILES = 32
CHUNKS = 80
PAIRS = CHUNKS // 2
HB = 40
EPT = CHUNKS * K
EP = TILES * EPT
ROWS_PT = NP // 16
DEGW = 128


def _zero_vmem(ref, nrows, ncols):
    per_row = ncols // 16

    def body(i, carry):
        ref[i // per_row, pl.ds((i % per_row) * 16, 16)] = jnp.zeros((16,), jnp.float32)
        return carry

    lax.fori_loop(0, nrows * per_row, body, 0)


def _fill_ones_vmem(ref, nrows, ncols):
    per_row = ncols // 16

    def body(i, carry):
        ref[i // per_row, pl.ds((i % per_row) * 16, 16)] = jnp.ones((16,), jnp.float32)
        return carry

    lax.fori_loop(0, nrows * per_row, body, 0)


@functools.lru_cache(maxsize=None)
def _sc_kernels():
    mesh = plsc.VectorSubcoreMesh(
        core_axis_name="c", subcore_axis_name="s", num_cores=2,
        num_subcores=16)

    @functools.partial(
        pl.kernel,
        out_type=jax.ShapeDtypeStruct((2, NP, DEGW), jnp.float32),
        mesh=mesh,
        scratch_types=[
            pltpu.VMEM((CHUNKS, K), jnp.int32),
            pltpu.VMEM((K, DEGW), jnp.float32),
            pltpu.VMEM((K, DEGW), jnp.float32),
            pltpu.VMEM_SHARED((NP, DEGW), jnp.float32),
            pltpu.SemaphoreType.DMA,
            pltpu.SemaphoreType.DMA,
        ],
    )
    def deg_kernel(dst_hbm, out_hbm, didx2d, ones_v, zb, acc, sem0, sem1):
        cid = lax.axis_index("c")
        sid = lax.axis_index("s")
        wid = cid * 16 + sid

        pltpu.sync_copy(dst_hbm.at[pl.ds(wid * CHUNKS, CHUNKS), :], didx2d)
        _zero_vmem(zb, K, DEGW)
        _fill_ones_vmem(ones_v, K, DEGW)
        for t in range(ROWS_PT // K):
            pltpu.sync_copy(zb, acc.at[pl.ds(sid * ROWS_PT + t * K, K), :])
        plsc.subcore_barrier()

        def group(g, carry):
            for j in range(8):
                pltpu.async_copy(ones_v, acc.at[didx2d.at[g * 8 + j]], sem0,
                                 add=True)
            for j in range(8):
                pltpu.make_async_copy(ones_v, acc.at[didx2d.at[g * 8 + j]],
                                      sem0).wait()
            return carry

        lax.fori_loop(0, CHUNKS // 8, group, 0)
        plsc.subcore_barrier()
        for t in range(ROWS_PT // K):
            r = sid * ROWS_PT + t * K
            pltpu.sync_copy(acc.at[pl.ds(r, K), :], zb)
            pltpu.sync_copy(zb, out_hbm.at[cid, pl.ds(r, K), :])

    @functools.partial(
        pl.kernel,
        out_type=jax.ShapeDtypeStruct((2, NP, D), jnp.float32),
        mesh=mesh,
        scratch_types=[
            pltpu.VMEM((K,), jnp.int32),
            pltpu.VMEM((K,), jnp.int32),
            pltpu.VMEM((K, D), jnp.float32),
            pltpu.VMEM_SHARED((NP, D), jnp.float32),
            pltpu.SemaphoreType.DMA,
        ],
    )
    def agg_kernel(xt_hbm, src_hbm, dst_hbm, out_hbm, sidx, didx, rows0, acc,
                   gsem0):
        cid = lax.axis_index("c")
        sid = lax.axis_index("s")

        _zero_vmem(rows0, K, D)
        for t in range(ROWS_PT // K):
            pltpu.sync_copy(rows0, acc.at[pl.ds(sid * ROWS_PT + t * K, K), :])
        plsc.subcore_barrier()

        ebase = (cid * 16 + sid) * EPT

        def chunk(c, carry):
            b = ebase + c * K
            pltpu.sync_copy(src_hbm.at[pl.ds(b, K)], sidx)
            pltpu.sync_copy(dst_hbm.at[pl.ds(b, K)], didx)
            pltpu.async_copy(xt_hbm.at[sidx], rows0, gsem0).wait()
            pltpu.sync_copy(rows0, acc.at[didx], add=True)
            return carry

        lax.fori_loop(0, CHUNKS, chunk, 0)

        plsc.subcore_barrier()
        for t in range(ROWS_PT // K):
            r = sid * ROWS_PT + t * K
            pltpu.sync_copy(acc.at[pl.ds(r, K), :], rows0)
            pltpu.sync_copy(rows0, out_hbm.at[cid, pl.ds(r, K), :])

    return deg_kernel, agg_kernel


_RB = 400


def _prep_body(dp0_ref, dp1_ref, x_ref, dinv_ref, xt_ref):
    deg = dp0_ref[:, :1] + dp1_ref[:, :1] + 1.0
    dv = lax.rsqrt(jnp.maximum(deg, 1.0))
    dinv_ref[...] = dv
    xt_ref[...] = dv * x_ref[...]


def _tc_prep(degp, x):
    return pl.pallas_call(
        _prep_body,
        grid=(N // _RB,),
        in_specs=[
            pl.BlockSpec((_RB, DEGW), lambda i: (i, 0)),
            pl.BlockSpec((_RB, DEGW), lambda i: (i, 0)),
            pl.BlockSpec((_RB, D), lambda i: (i, 0)),
        ],
        out_specs=[
            pl.BlockSpec((_RB, 1), lambda i: (i, 0)),
            pl.BlockSpec((_RB, D), lambda i: (i, 0)),
        ],
        out_shape=[
            jax.ShapeDtypeStruct((N, 1), jnp.float32),
            jax.ShapeDtypeStruct((N, D), jnp.float32),
        ],
    )(degp[0], degp[1], x)


def _layer_body(relu, emit_xt, p0_ref, p1_ref, x_ref, dinv_ref, w_ref, b_ref,
                out_ref, xt_ref=None):
    dv = dinv_ref[...]
    z = dv * (p0_ref[...] + p1_ref[...]) + (dv * dv) * x_ref[...]
    h = jnp.dot(z, w_ref[...], preferred_element_type=jnp.float32) + b_ref[...]
    if relu:
        h = jnp.maximum(h, 0.0)
    out_ref[...] = h
    if emit_xt:
        xt_ref[...] = dv * h


def _tc_layer(p, x, dinv, w, b, relu, emit_xt):
    c = w.shape[1]
    out_shape = [jax.ShapeDtypeStruct((N, c), jnp.float32)]
    out_specs = [pl.BlockSpec((_RB, c), lambda i: (i, 0))]
    if emit_xt:
        out_shape.append(jax.ShapeDtypeStruct((N, c), jnp.float32))
        out_specs.append(pl.BlockSpec((_RB, c), lambda i: (i, 0)))
    res = pl.pallas_call(
        functools.partial(_layer_body, relu, emit_xt),
        grid=(N // _RB,),
        in_specs=[
            pl.BlockSpec((_RB, D), lambda i: (i, 0)),
            pl.BlockSpec((_RB, D), lambda i: (i, 0)),
            pl.BlockSpec((_RB, D), lambda i: (i, 0)),
            pl.BlockSpec((_RB, 1), lambda i: (i, 0)),
            pl.BlockSpec((D, c), lambda i: (0, 0)),
            pl.BlockSpec((1, c), lambda i: (0, 0)),
        ],
        out_specs=out_specs,
        out_shape=out_shape,
    )(p[0, :N], p[1, :N], x, dinv, w, b.reshape(1, c))
    return res if emit_xt else res[0]


def kernel(edge_emb_eq1, edge_index, W_enc0, b_enc0, W_enc1, b_enc1,
           W_sh, b_sh, W_mu, b_mu, W_lv, b_lv):
    src = jnp.concatenate(
        [edge_index[0], jnp.zeros((EP - E,), edge_index.dtype)])
    dst = jnp.concatenate(
        [edge_index[1], jnp.full((EP - E,), TRASH, edge_index.dtype)])
    dst2d = dst.reshape(TILES * CHUNKS, K)

    deg_kernel, agg_kernel = _sc_kernels()
    degp = deg_kernel(dst2d)
    dinv, xt0 = _tc_prep(degp, edge_emb_eq1)

    p1 = agg_kernel(xt0, src, dst)
    x1, xt1 = _tc_layer(p1, edge_emb_eq1, dinv, W_enc0, b_enc0, False, True)
    p2 = agg_kernel(xt1, src, dst)
    x2, xt2 = _tc_layer(p2, x1, dinv, W_enc1, b_enc1, False, True)
    p3 = agg_kernel(xt2, src, dst)
    x3, xt3 = _tc_layer(p3, x2, dinv, W_sh, b_sh, True, True)
    p4 = agg_kernel(xt3, src, dst)
    wcat = jnp.concatenate([W_mu, W_lv], axis=1)
    bcat = jnp.concatenate([b_mu, b_lv])
    out = _tc_layer(p4, x3, dinv, wcat, bcat, False, False)
    return (out[:, :D], out[:, D:])


# speedup vs baseline: 1.5422x; 1.5422x over previous
import functools

import jax
import jax.numpy as jnp
from jax import lax
from jax.experimental import pallas as pl
from jax.experimental.pallas import tpu as pltpu
from jax.experimental.pallas import tpu_sc as plsc

N = 10000
E = 320000
D = 128

NP = 10240
TRASH = 10000
K = 128
TILES = 32
CHUNKS = 79
EP = TILES * CHUNKS * K
EPT = CHUNKS * K
ROWS_PT = NP // TILES * 2
DEGW = 128


def _zero_vmem(ref, nrows, ncols):
    per_row = ncols // 16

    def body(i, carry):
        ref[i // per_row, pl.ds((i % per_row) * 16, 16)] = jnp.zeros((16,), jnp.float32)
        return carry

    lax.fori_loop(0, nrows * per_row, body, 0)


def _fill_ones_vmem(ref, nrows, ncols):
    per_row = ncols // 16

    def body(i, carry):
        ref[i // per_row, pl.ds((i % per_row) * 16, 16)] = jnp.ones((16,), jnp.float32)
        return carry

    lax.fori_loop(0, nrows * per_row, body, 0)


@functools.lru_cache(maxsize=None)
def _sc_kernels():
    mesh = plsc.VectorSubcoreMesh(
        core_axis_name="c", subcore_axis_name="s", num_cores=2,
        num_subcores=16)

    @functools.partial(
        pl.kernel,
        out_type=jax.ShapeDtypeStruct((2, NP, DEGW), jnp.float32),
        mesh=mesh,
        scratch_types=[
            pltpu.VMEM((K,), jnp.int32),
            pltpu.VMEM((K, DEGW), jnp.float32),
            pltpu.VMEM((64, DEGW), jnp.float32),
            pltpu.VMEM_SHARED((NP, DEGW), jnp.float32),
        ],
    )
    def deg_kernel(dst_hbm, out_hbm, didx, ones_v, zb, acc):
        cid = lax.axis_index("c")
        sid = lax.axis_index("s")

        _zero_vmem(zb, 64, DEGW)
        _fill_ones_vmem(ones_v, K, DEGW)
        for t in range(ROWS_PT // 64):
            pltpu.sync_copy(zb, acc.at[pl.ds(sid * ROWS_PT + t * 64, 64), :])
        plsc.subcore_barrier()

        ebase = (cid * 16 + sid) * EPT

        def chunk(c, carry):
            pltpu.sync_copy(dst_hbm.at[pl.ds(ebase + c * K, K)], didx)
            pltpu.sync_copy(ones_v, acc.at[didx], add=True)
            return carry

        lax.fori_loop(0, CHUNKS, chunk, 0)
        plsc.subcore_barrier()

        for t in range(ROWS_PT // 64):
            r = sid * ROWS_PT + t * 64
            pltpu.sync_copy(acc.at[pl.ds(r, 64), :], zb)
            pltpu.sync_copy(zb, out_hbm.at[cid, pl.ds(r, 64), :])

    @functools.partial(
        pl.kernel,
        out_type=jax.ShapeDtypeStruct((2, NP, D), jnp.float32),
        mesh=mesh,
        scratch_types=[
            pltpu.VMEM((K,), jnp.int32),
            pltpu.VMEM((K,), jnp.int32),
            pltpu.VMEM((K, D), jnp.float32),
            pltpu.VMEM((64, D), jnp.float32),
            pltpu.VMEM_SHARED((NP, D), jnp.float32),
            pltpu.SemaphoreType.DMA,
        ],
    )
    def agg_kernel(xt_hbm, src_hbm, dst_hbm, out_hbm, sidx, didx, rows, zb,
                   acc, sem):
        cid = lax.axis_index("c")
        sid = lax.axis_index("s")

        _zero_vmem(zb, 64, D)
        for t in range(ROWS_PT // 64):
            pltpu.sync_copy(zb, acc.at[pl.ds(sid * ROWS_PT + t * 64, 64), :])
        plsc.subcore_barrier()

        ebase = (cid * 16 + sid) * EPT

        def chunk(c, carry):
            b = ebase + c * K
            pltpu.sync_copy(src_hbm.at[pl.ds(b, K)], sidx)
            pltpu.sync_copy(dst_hbm.at[pl.ds(b, K)], didx)
            pltpu.async_copy(xt_hbm.at[sidx], rows, sem).wait()
            pltpu.sync_copy(rows, acc.at[didx], add=True)
            return carry

        lax.fori_loop(0, CHUNKS, chunk, 0)
        plsc.subcore_barrier()

        for t in range(ROWS_PT // K):
            r = sid * ROWS_PT + t * K
            pltpu.sync_copy(acc.at[pl.ds(r, K), :], rows)
            pltpu.sync_copy(rows, out_hbm.at[cid, pl.ds(r, K), :])

    return deg_kernel, agg_kernel


_RB = 400


def _prep_body(dp0_ref, dp1_ref, x_ref, dinv_ref, xt_ref):
    deg = dp0_ref[:, :1] + dp1_ref[:, :1] + 1.0
    dv = lax.rsqrt(jnp.maximum(deg, 1.0))
    dinv_ref[...] = dv
    xt_ref[...] = dv * x_ref[...]


def _tc_prep(degp, x):
    return pl.pallas_call(
        _prep_body,
        grid=(N // _RB,),
        in_specs=[
            pl.BlockSpec((_RB, DEGW), lambda i: (i, 0)),
            pl.BlockSpec((_RB, DEGW), lambda i: (i, 0)),
            pl.BlockSpec((_RB, D), lambda i: (i, 0)),
        ],
        out_specs=[
            pl.BlockSpec((_RB, 1), lambda i: (i, 0)),
            pl.BlockSpec((_RB, D), lambda i: (i, 0)),
        ],
        out_shape=[
            jax.ShapeDtypeStruct((N, 1), jnp.float32),
            jax.ShapeDtypeStruct((N, D), jnp.float32),
        ],
    )(degp[0], degp[1], x)


def _layer_body(relu, emit_xt, p0_ref, p1_ref, x_ref, dinv_ref, w_ref, b_ref,
                out_ref, xt_ref=None):
    dv = dinv_ref[...]
    z = dv * (p0_ref[...] + p1_ref[...]) + (dv * dv) * x_ref[...]
    h = jnp.dot(z, w_ref[...], preferred_element_type=jnp.float32) + b_ref[...]
    if relu:
        h = jnp.maximum(h, 0.0)
    out_ref[...] = h
    if emit_xt:
        xt_ref[...] = dv * h


def _tc_layer(p, x, dinv, w, b, relu, emit_xt):
    c = w.shape[1]
    out_shape = [jax.ShapeDtypeStruct((N, c), jnp.float32)]
    out_specs = [pl.BlockSpec((_RB, c), lambda i: (i, 0))]
    if emit_xt:
        out_shape.append(jax.ShapeDtypeStruct((N, c), jnp.float32))
        out_specs.append(pl.BlockSpec((_RB, c), lambda i: (i, 0)))
    res = pl.pallas_call(
        functools.partial(_layer_body, relu, emit_xt),
        grid=(N // _RB,),
        in_specs=[
            pl.BlockSpec((_RB, D), lambda i: (i, 0)),
            pl.BlockSpec((_RB, D), lambda i: (i, 0)),
            pl.BlockSpec((_RB, D), lambda i: (i, 0)),
            pl.BlockSpec((_RB, 1), lambda i: (i, 0)),
            pl.BlockSpec((D, c), lambda i: (0, 0)),
            pl.BlockSpec((1, c), lambda i: (0, 0)),
        ],
        out_specs=out_specs,
        out_shape=out_shape,
    )(p[0, :N], p[1, :N], x, dinv, w, b.reshape(1, c))
    return res if emit_xt else res[0]


def kernel(edge_emb_eq1, edge_index, W_enc0, b_enc0, W_enc1, b_enc1,
           W_sh, b_sh, W_mu, b_mu, W_lv, b_lv):
    src = jnp.concatenate(
        [edge_index[0], jnp.zeros((EP - E,), edge_index.dtype)])
    dst = jnp.concatenate(
        [edge_index[1], jnp.full((EP - E,), TRASH, edge_index.dtype)])

    deg_kernel, agg_kernel = _sc_kernels()
    degp = deg_kernel(dst)
    dinv, xt0 = _tc_prep(degp, edge_emb_eq1)

    p1 = agg_kernel(xt0, src, dst)
    x1, xt1 = _tc_layer(p1, edge_emb_eq1, dinv, W_enc0, b_enc0, False, True)
    p2 = agg_kernel(xt1, src, dst)
    x2, xt2 = _tc_layer(p2, x1, dinv, W_enc1, b_enc1, False, True)
    p3 = agg_kernel(xt2, src, dst)
    x3, xt3 = _tc_layer(p3, x2, dinv, W_sh, b_sh, True, True)
    p4 = agg_kernel(xt3, src, dst)
    wcat = jnp.concatenate([W_mu, W_lv], axis=1)
    bcat = jnp.concatenate([b_mu, b_lv])
    out = _tc_layer(p4, x3, dinv, wcat, bcat, False, False)
    return (out[:, :D], out[:, D:])
